# baseline (speedup 1.0000x reference)
import functools

import jax
import jax.numpy as jnp
from jax import lax
from jax.experimental import pallas as pl
from jax.experimental.pallas import tpu as pltpu
from jax.experimental.pallas import tpu_sc as plsc

_K = 8
_E = 64
_T = 1024
_NW = 32
_CHUNKS = (24, 8)

_SORT8 = [(0, 1), (2, 3), (4, 5), (6, 7),
          (0, 2), (1, 3), (4, 6), (5, 7),
          (1, 2), (5, 6),
          (0, 4), (1, 5), (2, 6), (3, 7),
          (2, 4), (3, 5),
          (1, 2), (3, 4), (5, 6)]
_BITONIC8 = [(0, 4), (1, 5), (2, 6), (3, 7),
             (0, 2), (1, 3), (4, 6), (5, 7),
             (0, 1), (2, 3), (4, 5), (6, 7)]


def _fused_block(x_ref, w_ref, scores_ref, idx_ref):
    x = x_ref[...]
    w = w_ref[...]
    logits = lax.dot_general(
        w, x, (((1,), (1,)), ((), ())), preferred_element_type=jnp.float32
    )
    m = jnp.max(logits, axis=0, keepdims=True)
    s = jnp.sum(jnp.exp(logits - m), axis=0, keepdims=True)
    rows = lax.broadcasted_iota(jnp.int32, logits.shape, 0)
    cur = logits
    svals = []
    sidx = []
    for _ in range(_K):
        mv = jnp.max(cur, axis=0, keepdims=True)
        ii = jnp.min(jnp.where(cur >= mv, rows, _E), axis=0, keepdims=True)
        svals.append(mv)
        sidx.append(ii)
        cur = jnp.where(rows == ii, -jnp.inf, cur)
    top = jnp.concatenate(svals, axis=0)
    scores_ref[...] = jnp.exp(top - m) / s
    idx_ref[...] = jnp.concatenate(sidx, axis=0)


def _probs_block(x_ref, w_ref, probs_ref):
    x = x_ref[...]
    w = w_ref[...]
    logits = lax.dot_general(
        w, x, (((1,), (1,)), ((), ())), preferred_element_type=jnp.float32
    )
    m = jnp.max(logits, axis=0, keepdims=True)
    e = jnp.exp(logits - m)
    s = jnp.sum(e, axis=0, keepdims=True)
    probs_ref[...] = e * (1.0 / s)


def _sc_top8(probs_hbm, scores_hbm, idx_hbm, buf, sco, sio):
    ntok = probs_hbm.shape[1]
    tw = ntok // _NW
    wid = lax.axis_index("s") * 2 + lax.axis_index("c")
    pltpu.sync_copy(probs_hbm.at[:, pl.ds(wid * tw, tw)], buf)
    lane = lax.iota(jnp.int32, 16)
    hi01 = lane >> 3
    kmask = [(lane & 7) == k for k in range(_K)]

    def group(g, carry):
        base = g * 16
        vs = [buf[j, pl.ds(base, 16)] for j in range(_E)]
        tops = []
        for grp in range(8):
            lst = [(vs[8 * grp + j], jnp.full((16,), 8 * grp + j, jnp.int32))
                   for j in range(8)]
            for (i, j) in _SORT8:
                av, ai = lst[i]
                bv, bi = lst[j]
                m = av >= bv
                lst[i] = (jnp.maximum(av, bv), jnp.where(m, ai, bi))
                lst[j] = (jnp.minimum(av, bv), jnp.where(m, bi, ai))
            tops.append(lst)
        while len(tops) > 1:
            nxt = []
            for t in range(0, len(tops), 2):
                a, b = tops[t], tops[t + 1]
                c = []
                for i in range(8):
                    av, ai = a[i]
                    bv, bi = b[7 - i]
                    c.append((jnp.maximum(av, bv),
                              jnp.where(av >= bv, ai, bi)))
                for (i, j) in _BITONIC8:
                    av, ai = c[i]
                    bv, bi = c[j]
                    m = av >= bv
                    c[i] = (jnp.maximum(av, bv), jnp.where(m, ai, bi))
                    c[j] = (jnp.minimum(av, bv), jnp.where(m, bi, ai))
                nxt.append(c)
            tops = nxt
        top = tops[0]
        for k in range(_K):
            sco[k, pl.ds(base, 16)] = top[k][0]
            sio[k, pl.ds(base, 16)] = top[k][1]
        return carry

    lax.fori_loop(0, tw // 16, group, 0)
    pltpu.sync_copy(sco, scores_hbm.at[wid])
    pltpu.sync_copy(sio, idx_hbm.at[wid])


@jax.jit
def kernel(x, W):
    n_tokens, emb = x.shape
    scores_parts = []
    idx_parts = []
    blk0 = 0
    for nb_c in _CHUNKS[:-1]:
        ntok = nb_c * _T
        tw = ntok // _NW
        probs_t = pl.pallas_call(
            _probs_block,
            grid=(nb_c,),
            in_specs=[
                pl.BlockSpec((_T, emb), lambda i, b=blk0: (b + i, 0)),
                pl.BlockSpec((_E, emb), lambda i: (0, 0)),
            ],
            out_specs=pl.BlockSpec((_E, _T), lambda i: (0, i)),
            out_shape=jax.ShapeDtypeStruct((_E, ntok), jnp.float32),
        )(x, W)

        sc_call = functools.partial(
            pl.kernel,
            out_type=[
                jax.ShapeDtypeStruct((_NW, _K, tw), jnp.float32),
                jax.ShapeDtypeStruct((_NW, _K, tw), jnp.int32),
            ],
            mesh=plsc.VectorSubcoreMesh(core_axis_name="c", subcore_axis_name="s"),
            scratch_types=[
                pltpu.VMEM((_E, tw), jnp.float32),
                pltpu.VMEM((_K, tw), jnp.float32),
                pltpu.VMEM((_K, tw), jnp.int32),
            ],
        )(_sc_top8)
        sco, sio = sc_call(probs_t)
        scores_parts.append(sco.transpose(0, 2, 1).reshape(-1, _K))
        idx_parts.append(sio.transpose(0, 2, 1).reshape(-1, _K))
        blk0 += nb_c

    nb_f = _CHUNKS[-1]
    ntok_f = nb_f * _T
    sco_f, sio_f = pl.pallas_call(
        _fused_block,
        grid=(nb_f,),
        in_specs=[
            pl.BlockSpec((_T, emb), lambda i, b=blk0: (b + i, 0)),
            pl.BlockSpec((_E, emb), lambda i: (0, 0)),
        ],
        out_specs=[
            pl.BlockSpec((_K, _T), lambda i: (0, i)),
            pl.BlockSpec((_K, _T), lambda i: (0, i)),
        ],
        out_shape=[
            jax.ShapeDtypeStruct((_K, ntok_f), jnp.float32),
            jax.ShapeDtypeStruct((_K, ntok_f), jnp.int32),
        ],
    )(x, W)
    scores_parts.append(sco_f.T)
    idx_parts.append(sio_f.T)

    return (jnp.concatenate(scores_parts, axis=0),
            jnp.concatenate(idx_parts, axis=0))


# speedup vs baseline: 1.0052x; 1.0052x over previous
import functools

import jax
import jax.numpy as jnp
from jax import lax
from jax.experimental import pallas as pl
from jax.experimental.pallas import tpu as pltpu
from jax.experimental.pallas import tpu_sc as plsc

_K = 8
_E = 64
_T = 1024
_NW = 32
_CHUNKS = (16, 16)

_SORT8 = [(0, 1), (2, 3), (4, 5), (6, 7),
          (0, 2), (1, 3), (4, 6), (5, 7),
          (1, 2), (5, 6),
          (0, 4), (1, 5), (2, 6), (3, 7),
          (2, 4), (3, 5),
          (1, 2), (3, 4), (5, 6)]
_BITONIC8 = [(0, 4), (1, 5), (2, 6), (3, 7),
             (0, 2), (1, 3), (4, 6), (5, 7),
             (0, 1), (2, 3), (4, 5), (6, 7)]


def _fused_block(x_ref, w_ref, scores_ref, idx_ref):
    x = x_ref[...]
    w = w_ref[...]
    logits = lax.dot_general(
        w, x, (((1,), (1,)), ((), ())), preferred_element_type=jnp.float32
    )
    m = jnp.max(logits, axis=0, keepdims=True)
    s = jnp.sum(jnp.exp(logits - m), axis=0, keepdims=True)
    rows = lax.broadcasted_iota(jnp.int32, logits.shape, 0)
    cur = logits
    svals = []
    sidx = []
    for _ in range(_K):
        mv = jnp.max(cur, axis=0, keepdims=True)
        ii = jnp.min(jnp.where(cur >= mv, rows, _E), axis=0, keepdims=True)
        svals.append(mv)
        sidx.append(ii)
        cur = jnp.where(rows == ii, -jnp.inf, cur)
    top = jnp.concatenate(svals, axis=0)
    scores_ref[...] = jnp.exp(top - m) / s
    idx_ref[...] = jnp.concatenate(sidx, axis=0)


def _probs_block(x_ref, w_ref, probs_ref):
    x = x_ref[...]
    w = w_ref[...]
    logits = lax.dot_general(
        w, x, (((1,), (1,)), ((), ())), preferred_element_type=jnp.float32
    )
    m = jnp.max(logits, axis=0, keepdims=True)
    e = jnp.exp(logits - m)
    s = jnp.sum(e, axis=0, keepdims=True)
    probs_ref[...] = e * (1.0 / s)


def _sc_top8(probs_hbm, scores_hbm, idx_hbm, buf, sco, sio):
    ntok = probs_hbm.shape[1]
    tw = ntok // _NW
    wid = lax.axis_index("s") * 2 + lax.axis_index("c")
    pltpu.sync_copy(probs_hbm.at[:, pl.ds(wid * tw, tw)], buf)
    lane = lax.iota(jnp.int32, 16)
    hi01 = lane >> 3
    kmask = [(lane & 7) == k for k in range(_K)]

    def group(g, carry):
        base = g * 16
        vs = [buf[j, pl.ds(base, 16)] for j in range(_E)]
        tops = []
        for grp in range(8):
            lst = [(vs[8 * grp + j], jnp.full((16,), 8 * grp + j, jnp.int32))
                   for j in range(8)]
            for (i, j) in _SORT8:
                av, ai = lst[i]
                bv, bi = lst[j]
                m = av >= bv
                lst[i] = (jnp.maximum(av, bv), jnp.where(m, ai, bi))
                lst[j] = (jnp.minimum(av, bv), jnp.where(m, bi, ai))
            tops.append(lst)
        while len(tops) > 1:
            nxt = []
            for t in range(0, len(tops), 2):
                a, b = tops[t], tops[t + 1]
                c = []
                for i in range(8):
                    av, ai = a[i]
                    bv, bi = b[7 - i]
                    c.append((jnp.maximum(av, bv),
                              jnp.where(av >= bv, ai, bi)))
                for (i, j) in _BITONIC8:
                    av, ai = c[i]
                    bv, bi = c[j]
                    m = av >= bv
                    c[i] = (jnp.maximum(av, bv), jnp.where(m, ai, bi))
                    c[j] = (jnp.minimum(av, bv), jnp.where(m, bi, ai))
                nxt.append(c)
            tops = nxt
        top = tops[0]
        for k in range(_K):
            sco[k, pl.ds(base, 16)] = top[k][0]
            sio[k, pl.ds(base, 16)] = top[k][1]
        return carry

    lax.fori_loop(0, tw // 16, group, 0)
    pltpu.sync_copy(sco, scores_hbm.at[wid])
    pltpu.sync_copy(sio, idx_hbm.at[wid])


@jax.jit
def kernel(x, W):
    n_tokens, emb = x.shape
    scores_parts = []
    idx_parts = []
    blk0 = 0
    for nb_c in _CHUNKS[:-1]:
        ntok = nb_c * _T
        tw = ntok // _NW
        probs_t = pl.pallas_call(
            _probs_block,
            grid=(nb_c,),
            in_specs=[
                pl.BlockSpec((_T, emb), lambda i, b=blk0: (b + i, 0)),
                pl.BlockSpec((_E, emb), lambda i: (0, 0)),
            ],
            out_specs=pl.BlockSpec((_E, _T), lambda i: (0, i)),
            out_shape=jax.ShapeDtypeStruct((_E, ntok), jnp.float32),
        )(x, W)

        sc_call = functools.partial(
            pl.kernel,
            out_type=[
                jax.ShapeDtypeStruct((_NW, _K, tw), jnp.float32),
                jax.ShapeDtypeStruct((_NW, _K, tw), jnp.int32),
            ],
            mesh=plsc.VectorSubcoreMesh(core_axis_name="c", subcore_axis_name="s"),
            scratch_types=[
                pltpu.VMEM((_E, tw), jnp.float32),
                pltpu.VMEM((_K, tw), jnp.float32),
                pltpu.VMEM((_K, tw), jnp.int32),
            ],
        )(_sc_top8)
        sco, sio = sc_call(probs_t)
        scores_parts.append(sco.transpose(0, 2, 1).reshape(-1, _K))
        idx_parts.append(sio.transpose(0, 2, 1).reshape(-1, _K))
        blk0 += nb_c

    nb_f = _CHUNKS[-1]
    ntok_f = nb_f * _T
    sco_f, sio_f = pl.pallas_call(
        _fused_block,
        grid=(nb_f,),
        in_specs=[
            pl.BlockSpec((_T, emb), lambda i, b=blk0: (b + i, 0)),
            pl.BlockSpec((_E, emb), lambda i: (0, 0)),
        ],
        out_specs=[
            pl.BlockSpec((_K, _T), lambda i: (0, i)),
            pl.BlockSpec((_K, _T), lambda i: (0, i)),
        ],
        out_shape=[
            jax.ShapeDtypeStruct((_K, ntok_f), jnp.float32),
            jax.ShapeDtypeStruct((_K, ntok_f), jnp.int32),
        ],
    )(x, W)
    scores_parts.append(sco_f.T)
    idx_parts.append(sio_f.T)

    return (jnp.concatenate(scores_parts, axis=0),
            jnp.concatenate(idx_parts, axis=0))


# speedup vs baseline: 1.0106x; 1.0054x over previous
import functools

import jax
import jax.numpy as jnp
from jax import lax
from jax.experimental import pallas as pl
from jax.experimental.pallas import tpu as pltpu
from jax.experimental.pallas import tpu_sc as plsc

_K = 8
_E = 64
_T = 1024
_NW = 32
_CHUNKS = (12, 20)

_SORT8 = [(0, 1), (2, 3), (4, 5), (6, 7),
          (0, 2), (1, 3), (4, 6), (5, 7),
          (1, 2), (5, 6),
          (0, 4), (1, 5), (2, 6), (3, 7),
          (2, 4), (3, 5),
          (1, 2), (3, 4), (5, 6)]
_BITONIC8 = [(0, 4), (1, 5), (2, 6), (3, 7),
             (0, 2), (1, 3), (4, 6), (5, 7),
             (0, 1), (2, 3), (4, 5), (6, 7)]


def _fused_block(x_ref, w_ref, scores_ref, idx_ref):
    x = x_ref[...]
    w = w_ref[...]
    logits = lax.dot_general(
        w, x, (((1,), (1,)), ((), ())), preferred_element_type=jnp.float32
    )
    m = jnp.max(logits, axis=0, keepdims=True)
    s = jnp.sum(jnp.exp(logits - m), axis=0, keepdims=True)
    rows = lax.broadcasted_iota(jnp.int32, logits.shape, 0)
    cur = logits
    svals = []
    sidx = []
    for _ in range(_K):
        mv = jnp.max(cur, axis=0, keepdims=True)
        ii = jnp.min(jnp.where(cur >= mv, rows, _E), axis=0, keepdims=True)
        svals.append(mv)
        sidx.append(ii)
        cur = jnp.where(rows == ii, -jnp.inf, cur)
    top = jnp.concatenate(svals, axis=0)
    scores_ref[...] = jnp.exp(top - m) / s
    idx_ref[...] = jnp.concatenate(sidx, axis=0)


def _probs_block(x_ref, w_ref, probs_ref):
    x = x_ref[...]
    w = w_ref[...]
    logits = lax.dot_general(
        w, x, (((1,), (1,)), ((), ())), preferred_element_type=jnp.float32
    )
    m = jnp.max(logits, axis=0, keepdims=True)
    e = jnp.exp(logits - m)
    s = jnp.sum(e, axis=0, keepdims=True)
    probs_ref[...] = e * (1.0 / s)


def _sc_top8(probs_hbm, scores_hbm, idx_hbm, buf, sco, sio):
    ntok = probs_hbm.shape[1]
    tw = ntok // _NW
    wid = lax.axis_index("s") * 2 + lax.axis_index("c")
    pltpu.sync_copy(probs_hbm.at[:, pl.ds(wid * tw, tw)], buf)
    lane = lax.iota(jnp.int32, 16)
    hi01 = lane >> 3
    kmask = [(lane & 7) == k for k in range(_K)]

    def group(g, carry):
        base = g * 16
        vs = [buf[j, pl.ds(base, 16)] for j in range(_E)]
        tops = []
        for grp in range(8):
            lst = [(vs[8 * grp + j], jnp.full((16,), 8 * grp + j, jnp.int32))
                   for j in range(8)]
            for (i, j) in _SORT8:
                av, ai = lst[i]
                bv, bi = lst[j]
                m = av >= bv
                lst[i] = (jnp.maximum(av, bv), jnp.where(m, ai, bi))
                lst[j] = (jnp.minimum(av, bv), jnp.where(m, bi, ai))
            tops.append(lst)
        while len(tops) > 1:
            nxt = []
            for t in range(0, len(tops), 2):
                a, b = tops[t], tops[t + 1]
                c = []
                for i in range(8):
                    av, ai = a[i]
                    bv, bi = b[7 - i]
                    c.append((jnp.maximum(av, bv),
                              jnp.where(av >= bv, ai, bi)))
                for (i, j) in _BITONIC8:
                    av, ai = c[i]
                    bv, bi = c[j]
                    m = av >= bv
                    c[i] = (jnp.maximum(av, bv), jnp.where(m, ai, bi))
                    c[j] = (jnp.minimum(av, bv), jnp.where(m, bi, ai))
                nxt.append(c)
            tops = nxt
        top = tops[0]
        for k in range(_K):
            sco[k, pl.ds(base, 16)] = top[k][0]
            sio[k, pl.ds(base, 16)] = top[k][1]
        return carry

    lax.fori_loop(0, tw // 16, group, 0)
    pltpu.sync_copy(sco, scores_hbm.at[wid])
    pltpu.sync_copy(sio, idx_hbm.at[wid])


@jax.jit
def kernel(x, W):
    n_tokens, emb = x.shape
    scores_parts = []
    idx_parts = []
    blk0 = 0
    for nb_c in _CHUNKS[:-1]:
        ntok = nb_c * _T
        tw = ntok // _NW
        probs_t = pl.pallas_call(
            _probs_block,
            grid=(nb_c,),
            in_specs=[
                pl.BlockSpec((_T, emb), lambda i, b=blk0: (b + i, 0)),
                pl.BlockSpec((_E, emb), lambda i: (0, 0)),
            ],
            out_specs=pl.BlockSpec((_E, _T), lambda i: (0, i)),
            out_shape=jax.ShapeDtypeStruct((_E, ntok), jnp.float32),
        )(x, W)

        sc_call = functools.partial(
            pl.kernel,
            out_type=[
                jax.ShapeDtypeStruct((_NW, _K, tw), jnp.float32),
                jax.ShapeDtypeStruct((_NW, _K, tw), jnp.int32),
            ],
            mesh=plsc.VectorSubcoreMesh(core_axis_name="c", subcore_axis_name="s"),
            scratch_types=[
                pltpu.VMEM((_E, tw), jnp.float32),
                pltpu.VMEM((_K, tw), jnp.float32),
                pltpu.VMEM((_K, tw), jnp.int32),
            ],
        )(_sc_top8)
        sco, sio = sc_call(probs_t)
        scores_parts.append(sco.transpose(0, 2, 1).reshape(-1, _K))
        idx_parts.append(sio.transpose(0, 2, 1).reshape(-1, _K))
        blk0 += nb_c

    nb_f = _CHUNKS[-1]
    ntok_f = nb_f * _T
    sco_f, sio_f = pl.pallas_call(
        _fused_block,
        grid=(nb_f,),
        in_specs=[
            pl.BlockSpec((_T, emb), lambda i, b=blk0: (b + i, 0)),
            pl.BlockSpec((_E, emb), lambda i: (0, 0)),
        ],
        out_specs=[
            pl.BlockSpec((_K, _T), lambda i: (0, i)),
            pl.BlockSpec((_K, _T), lambda i: (0, i)),
        ],
        out_shape=[
            jax.ShapeDtypeStruct((_K, ntok_f), jnp.float32),
            jax.ShapeDtypeStruct((_K, ntok_f), jnp.int32),
        ],
    )(x, W)
    scores_parts.append(sco_f.T)
    idx_parts.append(sio_f.T)

    return (jnp.concatenate(scores_parts, axis=0),
            jnp.concatenate(idx_parts, axis=0))
